# Initial kernel scaffold; baseline (speedup 1.0000x reference)
import jax
import jax.numpy as jnp
from jax.experimental import pallas as pl


def kernel(x, Wr, W1, b1, W2, b2, W3, b3):
    raise NotImplementedError("write your pallas kernel here")


# speedup vs baseline: 1.0798x; 1.0798x over previous
import functools

import jax
import jax.numpy as jnp
from jax.experimental import pallas as pl
from jax.experimental.pallas import tpu as pltpu

B, S, DIM = 1, 2048, 768
FFN = int(DIM * 2.0)
E, K = 8, 2
T = B * S
BT = 256
NI = T // BT


def _moe_dense_kernel(x_ref, wr_ref, w1_ref, b1_ref, w2_ref, b2_ref,
                      w3_ref, b3_ref, out_ref, comb_ref):
    e = pl.program_id(0)
    i = pl.program_id(1)

    x_t = x_ref[...]

    @pl.when(e == 0)
    def _router():
        logits = jnp.dot(x_t, wr_ref[...],
                         preferred_element_type=jnp.float32)
        m = jnp.max(logits, axis=-1, keepdims=True)
        ex = jnp.exp(logits - m)
        probs = ex / jnp.sum(ex, axis=-1, keepdims=True)
        col = jax.lax.broadcasted_iota(jnp.int32, probs.shape, 1)
        v1 = jnp.max(probs, axis=-1, keepdims=True)
        i1 = jnp.min(jnp.where(probs == v1, col, E), axis=-1, keepdims=True)
        probs2 = jnp.where(col == i1, -jnp.inf, probs)
        v2 = jnp.max(probs2, axis=-1, keepdims=True)
        i2 = jnp.min(jnp.where(probs2 == v2, col, E), axis=-1, keepdims=True)
        combine = jnp.where(col == i1, v1, 0.0) + jnp.where(col == i2, v2, 0.0)
        comb_ref[pl.ds(i * BT, BT), :] = combine

    cw_full = comb_ref[pl.ds(i * BT, BT), :]
    ecol = jax.lax.broadcasted_iota(jnp.int32, cw_full.shape, 1)
    cw = jnp.sum(jnp.where(ecol == e, cw_full, 0.0), axis=-1, keepdims=True)

    h1 = jnp.dot(x_t, w1_ref[0], preferred_element_type=jnp.float32) + b1_ref[0]
    h2 = jnp.dot(x_t, w2_ref[0], preferred_element_type=jnp.float32) + b2_ref[0]
    h = h1 * (1.0 / (1.0 + jnp.exp(-h1))) * h2
    y = jnp.dot(h, w3_ref[0], preferred_element_type=jnp.float32) + b3_ref[0]
    contrib = y * cw

    rows = pl.ds(i * BT, BT)

    @pl.when(e == 0)
    def _init():
        out_ref[rows, :] = contrib

    @pl.when(e > 0)
    def _acc():
        out_ref[rows, :] += contrib


def kernel(x, Wr, W1, b1, W2, b2, W3, b3):
    xf = jnp.transpose(x, (1, 0, 2)).reshape(T, DIM)

    out = pl.pallas_call(
        _moe_dense_kernel,
        grid=(E, NI),
        in_specs=[
            pl.BlockSpec((BT, DIM), lambda e, i: (i, 0)),
            pl.BlockSpec((DIM, E), lambda e, i: (0, 0)),
            pl.BlockSpec((1, DIM, FFN), lambda e, i: (e, 0, 0)),
            pl.BlockSpec((1, 1, FFN), lambda e, i: (e, 0, 0)),
            pl.BlockSpec((1, DIM, FFN), lambda e, i: (e, 0, 0)),
            pl.BlockSpec((1, 1, FFN), lambda e, i: (e, 0, 0)),
            pl.BlockSpec((1, FFN, DIM), lambda e, i: (e, 0, 0)),
            pl.BlockSpec((1, 1, DIM), lambda e, i: (e, 0, 0)),
        ],
        out_specs=pl.BlockSpec((T, DIM), lambda e, i: (0, 0)),
        out_shape=jax.ShapeDtypeStruct((T, DIM), jnp.float32),
        scratch_shapes=[pltpu.VMEM((T, E), jnp.float32)],
        compiler_params=pltpu.CompilerParams(
            dimension_semantics=("arbitrary", "arbitrary"),
        ),
    )(xf, Wr, W1, b1[:, None, :], W2, b2[:, None, :], W3, b3[:, None, :])

    return jnp.transpose(out.reshape(S, B, DIM), (1, 0, 2))


# speedup vs baseline: 1.1213x; 1.0384x over previous
import functools

import jax
import jax.numpy as jnp
from jax import lax
from jax.experimental import pallas as pl
from jax.experimental.pallas import tpu as pltpu
from jax.experimental.pallas import tpu_sc as plsc

B, S, DIM = 1, 2048, 768
FFN = int(DIM * 2.0)
E, K = 8, 2
T = B * S
BT = 256
NI = T // BT
BLK = 256
NTILES = (T * K + E * (BLK - 1) + BLK - 1) // BLK
P = NTILES * BLK

NC, NS = 2, 16
NW = NC * NS
JW = (T * K) // NW
TW = T // NW


def _lane_select(mat, idx):
    col = lax.broadcasted_iota(jnp.int32, mat.shape, 1)
    return jnp.sum(jnp.where(col == idx, mat, 0.0), axis=1, keepdims=True)


def _meta_kernel(x_ref, wr_ref, dest_ref, wts_ref, tile_e_ref,
                 carry_ref, meta_ref, base_ref):
    ph = pl.program_id(0)
    i = pl.program_id(1)
    rows = pl.ds(i * BT, BT)

    @pl.when(ph == 0)
    def _phase0():
        x_t = x_ref[...]
        logits = jnp.dot(x_t, wr_ref[...], preferred_element_type=jnp.float32)
        m = jnp.max(logits, axis=-1, keepdims=True)
        ex = jnp.exp(logits - m)
        probs = ex / jnp.sum(ex, axis=-1, keepdims=True)
        col = lax.broadcasted_iota(jnp.int32, probs.shape, 1)
        v1 = jnp.max(probs, axis=-1, keepdims=True)
        i1 = jnp.min(jnp.where(probs == v1, col, E), axis=-1, keepdims=True)
        probs2 = jnp.where(col == i1, -jnp.inf, probs)
        v2 = jnp.max(probs2, axis=-1, keepdims=True)
        i2 = jnp.min(jnp.where(probs2 == v2, col, E), axis=-1, keepdims=True)

        onehot = ((col == i1) | (col == i2)).astype(jnp.float32)

        @pl.when(i == 0)
        def _init():
            carry_ref[...] = jnp.zeros_like(carry_ref)

        ri = lax.broadcasted_iota(jnp.int32, (BT, BT), 0)
        cj = lax.broadcasted_iota(jnp.int32, (BT, BT), 1)
        ltri = (cj < ri).astype(jnp.float32)
        cex = jnp.dot(ltri, onehot, preferred_element_type=jnp.float32)
        cex = cex + carry_ref[...]
        carry_ref[...] += jnp.sum(onehot, axis=0, keepdims=True)

        r0 = _lane_select(cex, i1)
        r1 = _lane_select(cex, i2)
        meta_ref[rows, :] = jnp.concatenate(
            [r0, r1, i1.astype(jnp.float32), i2.astype(jnp.float32), v1, v2,
             jnp.zeros((BT, 2), jnp.float32)], axis=1)

    @pl.when(ph == 1)
    def _phase1():
        @pl.when(i == 0)
        def _bases():
            c = carry_ref[...]
            pc = jnp.floor((c + (BLK - 1)) / BLK) * BLK
            e1 = lax.broadcasted_iota(jnp.int32, (E, E), 0)
            e2 = lax.broadcasted_iota(jnp.int32, (E, E), 1)
            l8 = (e1 < e2).astype(jnp.float32)
            base_ref[...] = jnp.dot(pc, l8, preferred_element_type=jnp.float32)
            mm = lax.broadcasted_iota(jnp.int32, (NTILES, E), 0) * BLK
            cmp = (base_ref[...] <= mm.astype(jnp.float32)).astype(jnp.int32)
            te = jnp.sum(cmp, axis=1, keepdims=True) - 1
            tile_e_ref[...] = jnp.clip(te, 0, E - 1)

        mrow = meta_ref[rows, :]
        col8 = lax.broadcasted_iota(jnp.int32, mrow.shape, 1)

        def getc(c):
            return jnp.sum(jnp.where(col8 == c, mrow, 0.0), axis=1,
                           keepdims=True)

        r0, r1 = getc(0), getc(1)
        i1, i2 = getc(2).astype(jnp.int32), getc(3).astype(jnp.int32)
        v1, v2 = getc(4), getc(5)
        bases = jnp.broadcast_to(base_ref[...], (BT, E))
        d0 = _lane_select(bases, i1) + r0
        d1 = _lane_select(bases, i2) + r1
        dest_ref[...] = jnp.concatenate([d0, d1], axis=1).astype(jnp.int32)
        wts_ref[...] = jnp.concatenate([v1, v2], axis=1)


def _run_meta(xf, Wr):
    return pl.pallas_call(
        _meta_kernel,
        grid=(2, NI),
        in_specs=[
            pl.BlockSpec((BT, DIM), lambda p, i: (i, 0)),
            pl.BlockSpec((DIM, E), lambda p, i: (0, 0)),
        ],
        out_specs=[
            pl.BlockSpec((BT, K), lambda p, i: (i, 0)),
            pl.BlockSpec((BT, K), lambda p, i: (i, 0)),
            pl.BlockSpec((NTILES, 1), lambda p, i: (0, 0)),
        ],
        out_shape=[
            jax.ShapeDtypeStruct((T, K), jnp.int32),
            jax.ShapeDtypeStruct((T, K), jnp.float32),
            jax.ShapeDtypeStruct((NTILES, 1), jnp.int32),
        ],
        scratch_shapes=[
            pltpu.VMEM((1, E), jnp.float32),
            pltpu.VMEM((T, 8), jnp.float32),
            pltpu.VMEM((1, E), jnp.float32),
        ],
        compiler_params=pltpu.CompilerParams(
            dimension_semantics=("arbitrary", "arbitrary"),
        ),
    )(xf, Wr)


def _gemm_kernel(te_ref, xs_ref, w1_ref, b1_ref, w2_ref, b2_ref,
                 w3_ref, b3_ref, rw_ref, out_ref):
    x_t = xs_ref[...]
    h1 = jnp.dot(x_t, w1_ref[0], preferred_element_type=jnp.float32) + b1_ref[0]
    h2 = jnp.dot(x_t, w2_ref[0], preferred_element_type=jnp.float32) + b2_ref[0]
    h = h1 * (1.0 / (1.0 + jnp.exp(-h1))) * h2
    y = jnp.dot(h, w3_ref[0], preferred_element_type=jnp.float32) + b3_ref[0]
    out_ref[...] = y * rw_ref[...]


def _run_gemm(tile_e, xs, rw, W1, b1, W2, b2, W3, b3):
    grid_spec = pltpu.PrefetchScalarGridSpec(
        num_scalar_prefetch=1,
        grid=(NTILES,),
        in_specs=[
            pl.BlockSpec((BLK, DIM), lambda m, te: (m, 0)),
            pl.BlockSpec((1, DIM, FFN), lambda m, te: (te[m], 0, 0)),
            pl.BlockSpec((1, 1, FFN), lambda m, te: (te[m], 0, 0)),
            pl.BlockSpec((1, DIM, FFN), lambda m, te: (te[m], 0, 0)),
            pl.BlockSpec((1, 1, FFN), lambda m, te: (te[m], 0, 0)),
            pl.BlockSpec((1, FFN, DIM), lambda m, te: (te[m], 0, 0)),
            pl.BlockSpec((1, 1, DIM), lambda m, te: (te[m], 0, 0)),
            pl.BlockSpec((BLK, 1), lambda m, te: (m, 0)),
        ],
        out_specs=pl.BlockSpec((BLK, DIM), lambda m, te: (m, 0)),
    )
    return pl.pallas_call(
        _gemm_kernel,
        grid_spec=grid_spec,
        out_shape=jax.ShapeDtypeStruct((P, DIM), jnp.float32),
        compiler_params=pltpu.CompilerParams(
            dimension_semantics=("arbitrary",),
        ),
    )(tile_e, xs, W1, b1[:, None, :], W2, b2[:, None, :], W3, b3[:, None, :],
      rw)


def _make_dispatch():
    mesh = plsc.VectorSubcoreMesh(core_axis_name="c", subcore_axis_name="s")

    @functools.partial(
        pl.kernel, mesh=mesh,
        out_type=[
            jax.ShapeDtypeStruct((P, DIM), jnp.float32),
            jax.ShapeDtypeStruct((P,), jnp.float32),
        ],
        scratch_types=[
            pltpu.VMEM((JW,), jnp.int32),
            pltpu.VMEM((JW,), jnp.int32),
            pltpu.VMEM((JW,), jnp.float32),
            pltpu.VMEM((JW, DIM), jnp.float32),
            pltpu.SemaphoreType.DMA,
        ],
    )
    def dispatch(dest_hbm, tok_hbm, w_hbm, x_hbm, xs_hbm, rw_hbm,
                 dest_v, tok_v, w_v, rows_v, sem):
        wid = lax.axis_index("s") * NC + lax.axis_index("c")
        base = wid * JW
        pltpu.sync_copy(dest_hbm.at[pl.ds(base, JW)], dest_v)
        pltpu.sync_copy(tok_hbm.at[pl.ds(base, JW)], tok_v)
        pltpu.sync_copy(w_hbm.at[pl.ds(base, JW)], w_v)
        pltpu.async_copy(x_hbm.at[tok_v], rows_v, sem).wait()
        pltpu.async_copy(rows_v, xs_hbm.at[dest_v], sem).wait()
        pltpu.async_copy(w_v, rw_hbm.at[dest_v], sem).wait()

    return dispatch


def _make_combine():
    mesh = plsc.VectorSubcoreMesh(core_axis_name="c", subcore_axis_name="s")

    @functools.partial(
        pl.kernel, mesh=mesh,
        out_type=jax.ShapeDtypeStruct((T, DIM), jnp.float32),
        scratch_types=[
            pltpu.VMEM((TW,), jnp.int32),
            pltpu.VMEM((TW,), jnp.int32),
            pltpu.VMEM((TW, DIM), jnp.float32),
            pltpu.VMEM((TW, DIM), jnp.float32),
            pltpu.SemaphoreType.DMA,
        ],
    )
    def combine(d0_hbm, d1_hbm, yw_hbm, out_hbm, d0_v, d1_v, a_v, b_v, sem):
        wid = lax.axis_index("s") * NC + lax.axis_index("c")
        base = wid * TW
        pltpu.sync_copy(d0_hbm.at[pl.ds(base, TW)], d0_v)
        pltpu.sync_copy(d1_hbm.at[pl.ds(base, TW)], d1_v)
        pltpu.async_copy(yw_hbm.at[d0_v], a_v, sem).wait()
        pltpu.async_copy(yw_hbm.at[d1_v], b_v, sem).wait()

        def body(r, _):
            for c in range(DIM // 16):
                sl = pl.ds(c * 16, 16)
                a_v[r, sl] = a_v[r, sl] + b_v[r, sl]
            return 0

        lax.fori_loop(0, TW, body, 0)
        pltpu.sync_copy(a_v, out_hbm.at[pl.ds(base, TW)])

    return combine


_SC_KERNELS = {}


def _sc_kernels():
    if "dispatch" not in _SC_KERNELS:
        _SC_KERNELS["dispatch"] = _make_dispatch()
        _SC_KERNELS["combine"] = _make_combine()
    return _SC_KERNELS["dispatch"], _SC_KERNELS["combine"]


def kernel(x, Wr, W1, b1, W2, b2, W3, b3):
    dispatch, combine = _sc_kernels()
    xf = jnp.transpose(x, (1, 0, 2)).reshape(T, DIM)

    dest, wts, tile_e = _run_meta(xf, Wr)

    dest_flat = dest.reshape(T * K)
    w_flat = wts.reshape(T * K)
    tok_flat = jnp.repeat(jnp.arange(T, dtype=jnp.int32), K)

    xs, rw = dispatch(dest_flat, tok_flat, w_flat, xf)

    yw = _run_gemm(tile_e.reshape(NTILES), xs, rw.reshape(P, 1),
                   W1, b1, W2, b2, W3, b3)

    out = combine(dest[:, 0], dest[:, 1], yw)

    return jnp.transpose(out.reshape(S, B, DIM), (1, 0, 2))


# speedup vs baseline: 1.3924x; 1.2418x over previous
import functools

import jax
import jax.numpy as jnp
from jax import lax
from jax.experimental import pallas as pl
from jax.experimental.pallas import tpu as pltpu
from jax.experimental.pallas import tpu_sc as plsc

B, S, DIM = 1, 2048, 768
FFN = int(DIM * 2.0)
E, K = 8, 2
T = B * S
BT = 256
NI = T // BT
BLK = 256
NTILES = (T * K + E * (BLK - 1) + BLK - 1) // BLK
P = NTILES * BLK

NC, NS, L = 2, 16, 16
NW = NC * NS
TW = T // NW


def _lane_select(mat, idx):
    col = lax.broadcasted_iota(jnp.int32, mat.shape, 1)
    return jnp.sum(jnp.where(col == idx, mat, 0.0), axis=1, keepdims=True)


def _meta_kernel(x_ref, wr_ref, dest_ref, w0_ref, w1_ref, tile_e_ref,
                 carry_ref, meta_ref, base_ref):
    ph = pl.program_id(0)
    i = pl.program_id(1)
    rows = pl.ds(i * BT, BT)

    @pl.when(ph == 0)
    def _phase0():
        x_t = x_ref[...]
        logits = jnp.dot(x_t, wr_ref[...], preferred_element_type=jnp.float32)
        m = jnp.max(logits, axis=-1, keepdims=True)
        ex = jnp.exp(logits - m)
        probs = ex / jnp.sum(ex, axis=-1, keepdims=True)
        col = lax.broadcasted_iota(jnp.int32, probs.shape, 1)
        v1 = jnp.max(probs, axis=-1, keepdims=True)
        i1 = jnp.min(jnp.where(probs == v1, col, E), axis=-1, keepdims=True)
        probs2 = jnp.where(col == i1, -jnp.inf, probs)
        v2 = jnp.max(probs2, axis=-1, keepdims=True)
        i2 = jnp.min(jnp.where(probs2 == v2, col, E), axis=-1, keepdims=True)

        onehot = ((col == i1) | (col == i2)).astype(jnp.float32)

        @pl.when(i == 0)
        def _init():
            carry_ref[...] = jnp.zeros_like(carry_ref)

        ri = lax.broadcasted_iota(jnp.int32, (BT, BT), 0)
        cj = lax.broadcasted_iota(jnp.int32, (BT, BT), 1)
        ltri = (cj < ri).astype(jnp.float32)
        cex = jnp.dot(ltri, onehot, preferred_element_type=jnp.float32)
        cex = cex + carry_ref[...]
        carry_ref[...] += jnp.sum(onehot, axis=0, keepdims=True)

        r0 = _lane_select(cex, i1)
        r1 = _lane_select(cex, i2)
        meta_ref[rows, :] = jnp.concatenate(
            [r0, r1, i1.astype(jnp.float32), i2.astype(jnp.float32), v1, v2,
             jnp.zeros((BT, 2), jnp.float32)], axis=1)

    @pl.when(ph == 1)
    def _phase1():
        @pl.when(i == 0)
        def _bases():
            c = carry_ref[...]
            pc = jnp.floor((c + (BLK - 1)) / BLK) * BLK
            e1 = lax.broadcasted_iota(jnp.int32, (E, E), 0)
            e2 = lax.broadcasted_iota(jnp.int32, (E, E), 1)
            l8 = (e1 < e2).astype(jnp.float32)
            base_ref[...] = jnp.dot(pc, l8, preferred_element_type=jnp.float32)
            mm = lax.broadcasted_iota(jnp.int32, (NTILES, E), 0) * BLK
            cmp = (base_ref[...] <= mm.astype(jnp.float32)).astype(jnp.int32)
            te = jnp.sum(cmp, axis=1, keepdims=True) - 1
            tile_e_ref[...] = jnp.clip(te, 0, E - 1)

        mrow = meta_ref[rows, :]
        col8 = lax.broadcasted_iota(jnp.int32, mrow.shape, 1)

        def getc(c):
            return jnp.sum(jnp.where(col8 == c, mrow, 0.0), axis=1,
                           keepdims=True)

        r0, r1 = getc(0), getc(1)
        i1, i2 = getc(2).astype(jnp.int32), getc(3).astype(jnp.int32)
        v1, v2 = getc(4), getc(5)
        bases = jnp.broadcast_to(base_ref[...], (BT, E))
        d0 = _lane_select(bases, i1) + r0
        d1 = _lane_select(bases, i2) + r1
        dest_ref[...] = jnp.concatenate([d0, d1], axis=1).astype(jnp.int32)
        w0_ref[...] = jnp.broadcast_to(v1, (BT, L))
        w1_ref[...] = jnp.broadcast_to(v2, (BT, L))


def _run_meta(xf, Wr):
    return pl.pallas_call(
        _meta_kernel,
        grid=(2, NI),
        in_specs=[
            pl.BlockSpec((BT, DIM), lambda p, i: (i, 0)),
            pl.BlockSpec((DIM, E), lambda p, i: (0, 0)),
        ],
        out_specs=[
            pl.BlockSpec((BT, K), lambda p, i: (i, 0)),
            pl.BlockSpec((BT, L), lambda p, i: (i, 0)),
            pl.BlockSpec((BT, L), lambda p, i: (i, 0)),
            pl.BlockSpec((NTILES, 1), lambda p, i: (0, 0)),
        ],
        out_shape=[
            jax.ShapeDtypeStruct((T, K), jnp.int32),
            jax.ShapeDtypeStruct((T, L), jnp.float32),
            jax.ShapeDtypeStruct((T, L), jnp.float32),
            jax.ShapeDtypeStruct((NTILES, 1), jnp.int32),
        ],
        scratch_shapes=[
            pltpu.VMEM((1, E), jnp.float32),
            pltpu.VMEM((T, 8), jnp.float32),
            pltpu.VMEM((1, E), jnp.float32),
        ],
        compiler_params=pltpu.CompilerParams(
            dimension_semantics=("arbitrary", "arbitrary"),
        ),
    )(xf, Wr)


def _gemm_kernel(te_ref, xs_ref, w1_ref, b1_ref, w2_ref, b2_ref,
                 w3_ref, b3_ref, out_ref):
    x_t = xs_ref[...]
    h1 = jnp.dot(x_t, w1_ref[0], preferred_element_type=jnp.float32) + b1_ref[0]
    h2 = jnp.dot(x_t, w2_ref[0], preferred_element_type=jnp.float32) + b2_ref[0]
    h = h1 * (1.0 / (1.0 + jnp.exp(-h1))) * h2
    y = jnp.dot(h, w3_ref[0], preferred_element_type=jnp.float32) + b3_ref[0]
    out_ref[...] = y


def _run_gemm(tile_e, xs, W1, b1, W2, b2, W3, b3):
    grid_spec = pltpu.PrefetchScalarGridSpec(
        num_scalar_prefetch=1,
        grid=(NTILES,),
        in_specs=[
            pl.BlockSpec((BLK, DIM), lambda m, te: (m, 0)),
            pl.BlockSpec((1, DIM, FFN), lambda m, te: (te[m], 0, 0)),
            pl.BlockSpec((1, 1, FFN), lambda m, te: (te[m], 0, 0)),
            pl.BlockSpec((1, DIM, FFN), lambda m, te: (te[m], 0, 0)),
            pl.BlockSpec((1, 1, FFN), lambda m, te: (te[m], 0, 0)),
            pl.BlockSpec((1, FFN, DIM), lambda m, te: (te[m], 0, 0)),
            pl.BlockSpec((1, 1, DIM), lambda m, te: (te[m], 0, 0)),
        ],
        out_specs=pl.BlockSpec((BLK, DIM), lambda m, te: (m, 0)),
    )
    return pl.pallas_call(
        _gemm_kernel,
        grid_spec=grid_spec,
        out_shape=jax.ShapeDtypeStruct((P, DIM), jnp.float32),
        compiler_params=pltpu.CompilerParams(
            dimension_semantics=("arbitrary",),
        ),
    )(tile_e, xs, W1, b1[:, None, :], W2, b2[:, None, :], W3, b3[:, None, :])


def _make_dispatch():
    mesh = plsc.VectorSubcoreMesh(core_axis_name="c", subcore_axis_name="s")

    @functools.partial(
        pl.kernel, mesh=mesh,
        out_type=jax.ShapeDtypeStruct((P, DIM), jnp.float32),
        scratch_types=[
            pltpu.VMEM((TW,), jnp.int32),
            pltpu.VMEM((TW,), jnp.int32),
            pltpu.VMEM((TW, DIM), jnp.float32),
            pltpu.SemaphoreType.DMA,
        ],
    )
    def dispatch(d0_hbm, d1_hbm, x_hbm, xs_hbm, d0_v, d1_v, rows_v, sem):
        wid = lax.axis_index("s") * NC + lax.axis_index("c")
        base = wid * TW
        pltpu.sync_copy(d0_hbm.at[pl.ds(base, TW)], d0_v)
        pltpu.sync_copy(d1_hbm.at[pl.ds(base, TW)], d1_v)
        pltpu.sync_copy(x_hbm.at[pl.ds(base, TW)], rows_v)
        c0 = pltpu.async_copy(rows_v, xs_hbm.at[d0_v], sem)
        c1 = pltpu.async_copy(rows_v, xs_hbm.at[d1_v], sem)
        c0.wait()
        c1.wait()

    return dispatch


def _make_combine():
    mesh = plsc.VectorSubcoreMesh(core_axis_name="c", subcore_axis_name="s")

    @functools.partial(
        pl.kernel, mesh=mesh,
        out_type=jax.ShapeDtypeStruct((T, DIM), jnp.float32),
        scratch_types=[
            pltpu.VMEM((TW,), jnp.int32),
            pltpu.VMEM((TW,), jnp.int32),
            pltpu.VMEM((TW, L), jnp.float32),
            pltpu.VMEM((TW, L), jnp.float32),
            pltpu.VMEM((TW, DIM), jnp.float32),
            pltpu.VMEM((TW, DIM), jnp.float32),
            pltpu.SemaphoreType.DMA,
        ],
    )
    def combine(d0_hbm, d1_hbm, w0_hbm, w1_hbm, y_hbm, out_hbm,
                d0_v, d1_v, w0_v, w1_v, a_v, b_v, sem):
        wid = lax.axis_index("s") * NC + lax.axis_index("c")
        base = wid * TW
        pltpu.sync_copy(d0_hbm.at[pl.ds(base, TW)], d0_v)
        pltpu.sync_copy(d1_hbm.at[pl.ds(base, TW)], d1_v)
        pltpu.sync_copy(w0_hbm.at[pl.ds(base, TW)], w0_v)
        pltpu.sync_copy(w1_hbm.at[pl.ds(base, TW)], w1_v)
        c0 = pltpu.async_copy(y_hbm.at[d0_v], a_v, sem)
        c1 = pltpu.async_copy(y_hbm.at[d1_v], b_v, sem)
        c0.wait()
        c1.wait()

        def body(r, _):
            w0 = w0_v[r, :]
            w1 = w1_v[r, :]
            for c in range(DIM // L):
                sl = pl.ds(c * L, L)
                a_v[r, sl] = a_v[r, sl] * w0 + b_v[r, sl] * w1
            return 0

        lax.fori_loop(0, TW, body, 0)
        pltpu.sync_copy(a_v, out_hbm.at[pl.ds(base, TW)])

    return combine


_SC_KERNELS = {}


def _sc_kernels():
    if "dispatch" not in _SC_KERNELS:
        _SC_KERNELS["dispatch"] = _make_dispatch()
        _SC_KERNELS["combine"] = _make_combine()
    return _SC_KERNELS["dispatch"], _SC_KERNELS["combine"]


def kernel(x, Wr, W1, b1, W2, b2, W3, b3):
    dispatch, combine = _sc_kernels()
    xf = jnp.transpose(x, (1, 0, 2)).reshape(T, DIM)

    dest, w0b, w1b, tile_e = _run_meta(xf, Wr)
    d0 = dest[:, 0]
    d1 = dest[:, 1]

    xs = dispatch(d0, d1, xf)

    y = _run_gemm(tile_e.reshape(NTILES), xs, W1, b1, W2, b2, W3, b3)

    out = combine(d0, d1, w0b, w1b, y)

    return jnp.transpose(out.reshape(S, B, DIM), (1, 0, 2))
